# baseline (speedup 1.0000x reference)
import jax
import jax.numpy as jnp
from jax import lax
from jax.experimental import pallas as pl
from jax.experimental.pallas import tpu as pltpu
from jax.experimental.pallas import tpu_sc as plsc

_BOX = 128
_NC = 32
_NB = 8
_NPTS = 20000
_CG = 4
_CHUNK = 4000
_NCHUNK = _NPTS // _CHUNK
_GROUPS = _CHUNK // 16
_PIX = _BOX * _BOX
_ROWPAD = _PIX + _BOX + 8


def _splat_body(pts_hbm, vals_hbm, out_hbm, pts_v, vals_v, acc_v, sems):
    cid = lax.axis_index("c")
    sid = lax.axis_index("s")
    wid = sid * 2 + cid

    def _row0(t):
        task = wid * 2 + t
        b = task // (_NC // _CG)
        cg = task % (_NC // _CG)
        return b, b * _NC + cg * _CG

    corner = [[acc_v.at[0, pl.ds(k * _ROWPAD + dy * _BOX, _PIX + 8)]
               for k in range(_CG)]
              for dy in (0, 1)]

    def _start(t, ci):
        b, row0 = _row0(t)
        n0 = ci * _CHUNK
        slot = (t * _NCHUNK + ci) % 2
        cp = pltpu.make_async_copy(
            pts_hbm.at[pl.ds(b * 2, 2), pl.ds(n0, _CHUNK)],
            pts_v.at[slot], sems.at[slot])
        cv = pltpu.make_async_copy(
            vals_hbm.at[pl.ds(row0, _CG), pl.ds(n0, _CHUNK)],
            vals_v.at[slot], sems.at[slot])
        cp.start()
        cv.start()
        return cp, cv

    pend = _start(0, 0)
    for t in range(2):
        b, row0 = _row0(t)

        @plsc.parallel_loop(0, (_CG * _ROWPAD) // 16, unroll=10)
        def _zero(i):
            acc_v[0, pl.ds(i * 16, 16)] = jnp.zeros((16,), jnp.float32)

        for ci in range(_NCHUNK):
            if ci + 1 < _NCHUNK:
                nxt = _start(t, ci + 1)
            elif t == 0:
                nxt = _start(1, 0)
            else:
                nxt = None
            pend[0].wait()
            pend[1].wait()
            pend = nxt
            slot = (t * _NCHUNK + ci) % 2
            pts_c = pts_v.at[slot]
            vals_c = vals_v.at[slot]

            @plsc.parallel_loop(0, _GROUPS, unroll=5)
            def _grp(j):
                s = j * 16
                px = pts_c[0, pl.ds(s, 16)]
                py = pts_c[1, pl.ds(s, 16)]
                fx = (px + 0.5) * float(_BOX)
                fy = (py + 0.5) * float(_BOX)
                x0 = fx.astype(jnp.int32)
                y0 = fy.astype(jnp.int32)
                rx = fx - x0.astype(jnp.float32)
                ry = fy - y0.astype(jnp.float32)
                base = y0 * _BOX + x0
                idx = [base, base + 1]
                mx = x0 < (_BOX - 1)
                wgt = [1.0 - rx, rx]
                hgt = [1.0 - ry, ry]
                vals = [vals_c[k, pl.ds(s, 16)] for k in range(_CG)]
                for dy in (0, 1):
                    for dx in (0, 1):
                        w = wgt[dx] * hgt[dy]
                        msk = None if dx == 0 else mx
                        for k in range(_CG):
                            plsc.addupdate_scatter(
                                corner[dy][k], [idx[dx]], w * vals[k],
                                mask=msk)

        epis = [pltpu.make_async_copy(
                    acc_v.at[0, pl.ds(k * _ROWPAD, _PIX)],
                    out_hbm.at[pl.ds((row0 + k) * _PIX, _PIX)],
                    sems.at[2])
                for k in range(_CG)]
        for e in epis:
            e.start()
        for e in epis:
            e.wait()


@jax.jit
def _splat(pts2d, vals2d):
    mesh = plsc.VectorSubcoreMesh(core_axis_name="c", subcore_axis_name="s")
    run = pl.kernel(
        _splat_body,
        out_type=jax.ShapeDtypeStruct((_NB * _NC * _PIX,), jnp.float32),
        mesh=mesh,
        compiler_params=pltpu.CompilerParams(use_tc_tiling_on_sc=False,
                                             needs_layout_passes=False),
        scratch_types=[
            pltpu.VMEM((2, 2, _CHUNK), jnp.float32),
            pltpu.VMEM((2, _CG, _CHUNK), jnp.float32),
            pltpu.VMEM((1, _CG * _ROWPAD), jnp.float32),
            pltpu.SemaphoreType.DMA((3,)),
        ],
    )
    return run(pts2d, vals2d)


def kernel(points, values):
    pts2d = points.reshape(_NB * 2, _NPTS)
    vals2d = values.reshape(_NB * _NC, _NPTS)
    out = _splat(pts2d, vals2d)
    return out.reshape(_NB, _NC, _BOX, _BOX)


# speedup vs baseline: 1.0137x; 1.0137x over previous
import jax
import jax.numpy as jnp
from jax import lax
from jax.experimental import pallas as pl
from jax.experimental.pallas import tpu as pltpu
from jax.experimental.pallas import tpu_sc as plsc

_BOX = 128
_NC = 32
_NB = 8
_NPTS = 20000
_CG = 4
_CHUNK = 4000
_NCHUNK = _NPTS // _CHUNK
_GROUPS = _CHUNK // 16
_PIX = _BOX * _BOX
_ROWPAD = _PIX + _BOX + 8


def _splat_body(pts_hbm, vals_hbm, out_hbm, pts_v, vals_v, acc_v, sems):
    cid = lax.axis_index("c")
    sid = lax.axis_index("s")
    wid = sid * 2 + cid

    def _row0(t):
        task = wid * 2 + t
        b = task // (_NC // _CG)
        cg = task % (_NC // _CG)
        return b, b * _NC + cg * _CG

    corner = [[acc_v.at[0, pl.ds(k * _ROWPAD + dy * _BOX, _PIX + 8)]
               for k in range(_CG)]
              for dy in (0, 1)]

    def _start(t, ci):
        b, row0 = _row0(t)
        n0 = ci * _CHUNK
        slot = (t * _NCHUNK + ci) % 2
        cp = pltpu.make_async_copy(
            pts_hbm.at[pl.ds(b * 2, 2), pl.ds(n0, _CHUNK)],
            pts_v.at[slot], sems.at[slot])
        cv = pltpu.make_async_copy(
            vals_hbm.at[pl.ds(row0, _CG), pl.ds(n0, _CHUNK)],
            vals_v.at[slot], sems.at[slot])
        cp.start()
        cv.start()
        return cp, cv

    pend = _start(0, 0)
    for t in range(2):
        b, row0 = _row0(t)

        @plsc.parallel_loop(0, (_CG * _ROWPAD) // 16, unroll=10)
        def _zero(i):
            acc_v[0, pl.ds(i * 16, 16)] = jnp.zeros((16,), jnp.float32)

        for ci in range(_NCHUNK):
            if ci + 1 < _NCHUNK:
                nxt = _start(t, ci + 1)
            elif t == 0:
                nxt = _start(1, 0)
            else:
                nxt = None
            pend[0].wait()
            pend[1].wait()
            pend = nxt
            slot = (t * _NCHUNK + ci) % 2
            pts_c = pts_v.at[slot]
            vals_c = vals_v.at[slot]

            @plsc.parallel_loop(0, _GROUPS, unroll=2)
            def _grp(j):
                s = j * 16
                px = pts_c[0, pl.ds(s, 16)]
                py = pts_c[1, pl.ds(s, 16)]
                fx = (px + 0.5) * float(_BOX)
                fy = (py + 0.5) * float(_BOX)
                x0 = fx.astype(jnp.int32)
                y0 = fy.astype(jnp.int32)
                rx = fx - x0.astype(jnp.float32)
                ry = fy - y0.astype(jnp.float32)
                base = y0 * _BOX + x0
                idx = [base, base + 1]
                mx = x0 < (_BOX - 1)
                wgt = [1.0 - rx, rx]
                hgt = [1.0 - ry, ry]
                vals = [vals_c[k, pl.ds(s, 16)] for k in range(_CG)]
                for dy in (0, 1):
                    for dx in (0, 1):
                        w = wgt[dx] * hgt[dy]
                        msk = None if dx == 0 else mx
                        for k in range(_CG):
                            plsc.addupdate_scatter(
                                corner[dy][k], [idx[dx]], w * vals[k],
                                mask=msk)

        epis = [pltpu.make_async_copy(
                    acc_v.at[0, pl.ds(k * _ROWPAD, _PIX)],
                    out_hbm.at[pl.ds((row0 + k) * _PIX, _PIX)],
                    sems.at[2])
                for k in range(_CG)]
        for e in epis:
            e.start()
        for e in epis:
            e.wait()


@jax.jit
def _splat(pts2d, vals2d):
    mesh = plsc.VectorSubcoreMesh(core_axis_name="c", subcore_axis_name="s")
    run = pl.kernel(
        _splat_body,
        out_type=jax.ShapeDtypeStruct((_NB * _NC * _PIX,), jnp.float32),
        mesh=mesh,
        compiler_params=pltpu.CompilerParams(use_tc_tiling_on_sc=False,
                                             needs_layout_passes=False),
        scratch_types=[
            pltpu.VMEM((2, 2, _CHUNK), jnp.float32),
            pltpu.VMEM((2, _CG, _CHUNK), jnp.float32),
            pltpu.VMEM((1, _CG * _ROWPAD), jnp.float32),
            pltpu.SemaphoreType.DMA((3,)),
        ],
    )
    return run(pts2d, vals2d)


def kernel(points, values):
    pts2d = points.reshape(_NB * 2, _NPTS)
    vals2d = values.reshape(_NB * _NC, _NPTS)
    out = _splat(pts2d, vals2d)
    return out.reshape(_NB, _NC, _BOX, _BOX)


# speedup vs baseline: 1.0157x; 1.0019x over previous
import jax
import jax.numpy as jnp
from jax import lax
from jax.experimental import pallas as pl
from jax.experimental.pallas import tpu as pltpu
from jax.experimental.pallas import tpu_sc as plsc

_BOX = 128
_NC = 32
_NB = 8
_NPTS = 20000
_CG = 4
_CHUNK = 4000
_NCHUNK = _NPTS // _CHUNK
_GROUPS = _CHUNK // 16
_PIX = _BOX * _BOX
_ROWPAD = _PIX + _BOX + 8


def _splat_body(pts_hbm, vals_hbm, out_hbm, pts_v, vals_v, acc_v, sems):
    cid = lax.axis_index("c")
    sid = lax.axis_index("s")
    wid = sid * 2 + cid

    def _row0(t):
        task = wid * 2 + t
        b = task // (_NC // _CG)
        cg = task % (_NC // _CG)
        return b, b * _NC + cg * _CG

    corner = [[acc_v.at[0, pl.ds(k * _ROWPAD + dy * _BOX, _PIX + 8)]
               for k in range(_CG)]
              for dy in (0, 1)]

    def _start(t, ci):
        b, row0 = _row0(t)
        n0 = ci * _CHUNK
        slot = (t * _NCHUNK + ci) % 2
        cp = pltpu.make_async_copy(
            pts_hbm.at[pl.ds(b * 2, 2), pl.ds(n0, _CHUNK)],
            pts_v.at[slot], sems.at[slot])
        cv = pltpu.make_async_copy(
            vals_hbm.at[pl.ds(row0, _CG), pl.ds(n0, _CHUNK)],
            vals_v.at[slot], sems.at[slot])
        cp.start()
        cv.start()
        return cp, cv

    pend = _start(0, 0)
    epis = None
    for t in range(2):
        b, row0 = _row0(t)

        if epis is None:
            @plsc.parallel_loop(0, (_CG * _ROWPAD) // 16, unroll=10)
            def _zero(i):
                acc_v[0, pl.ds(i * 16, 16)] = jnp.zeros((16,), jnp.float32)
        else:
            for h in range(2):
                epis[2 * h].wait()
                epis[2 * h + 1].wait()
                h0 = h * (2 * _ROWPAD) // 16

                @plsc.parallel_loop(0, (2 * _ROWPAD) // 16, unroll=5)
                def _zero(i):
                    acc_v[0, pl.ds((h0 + i) * 16, 16)] = (
                        jnp.zeros((16,), jnp.float32))

        for ci in range(_NCHUNK):
            if ci + 1 < _NCHUNK:
                nxt = _start(t, ci + 1)
            elif t == 0:
                nxt = _start(1, 0)
            else:
                nxt = None
            pend[0].wait()
            pend[1].wait()
            pend = nxt
            slot = (t * _NCHUNK + ci) % 2
            pts_c = pts_v.at[slot]
            vals_c = vals_v.at[slot]

            @plsc.parallel_loop(0, _GROUPS, unroll=2)
            def _grp(j):
                s = j * 16
                px = pts_c[0, pl.ds(s, 16)]
                py = pts_c[1, pl.ds(s, 16)]
                fx = (px + 0.5) * float(_BOX)
                fy = (py + 0.5) * float(_BOX)
                x0 = fx.astype(jnp.int32)
                y0 = fy.astype(jnp.int32)
                rx = fx - x0.astype(jnp.float32)
                ry = fy - y0.astype(jnp.float32)
                base = y0 * _BOX + x0
                idx = [base, base + 1]
                mx = x0 < (_BOX - 1)
                wgt = [1.0 - rx, rx]
                hgt = [1.0 - ry, ry]
                vals = [vals_c[k, pl.ds(s, 16)] for k in range(_CG)]
                for dy in (0, 1):
                    for dx in (0, 1):
                        w = wgt[dx] * hgt[dy]
                        msk = None if dx == 0 else mx
                        for k in range(_CG):
                            plsc.addupdate_scatter(
                                corner[dy][k], [idx[dx]], w * vals[k],
                                mask=msk)

        epis = [pltpu.make_async_copy(
                    acc_v.at[0, pl.ds(k * _ROWPAD, _PIX)],
                    out_hbm.at[pl.ds((row0 + k) * _PIX, _PIX)],
                    sems.at[2])
                for k in range(_CG)]
        for e in epis:
            e.start()
    for e in epis:
        e.wait()


@jax.jit
def _splat(pts2d, vals2d):
    mesh = plsc.VectorSubcoreMesh(core_axis_name="c", subcore_axis_name="s")
    run = pl.kernel(
        _splat_body,
        out_type=jax.ShapeDtypeStruct((_NB * _NC * _PIX,), jnp.float32),
        mesh=mesh,
        compiler_params=pltpu.CompilerParams(use_tc_tiling_on_sc=False,
                                             needs_layout_passes=False),
        scratch_types=[
            pltpu.VMEM((2, 2, _CHUNK), jnp.float32),
            pltpu.VMEM((2, _CG, _CHUNK), jnp.float32),
            pltpu.VMEM((1, _CG * _ROWPAD), jnp.float32),
            pltpu.SemaphoreType.DMA((3,)),
        ],
    )
    return run(pts2d, vals2d)


def kernel(points, values):
    pts2d = points.reshape(_NB * 2, _NPTS)
    vals2d = values.reshape(_NB * _NC, _NPTS)
    out = _splat(pts2d, vals2d)
    return out.reshape(_NB, _NC, _BOX, _BOX)
